# baseline (speedup 1.0000x reference)
import jax
import jax.numpy as jnp
from jax import lax
from jax.experimental import pallas as pl
from jax.experimental.pallas import tpu as pltpu

N_BINS = 15
NPT = 100000
NCLS = 1000
BR = 1000
NBLK = NPT // BR
N_RANKS = 32
ONE_BITS = 0x3F800000


def _rowstats_kernel(x_ref, lab_ref, conf_ref, acc_ref):
    x = x_ref[...]
    m = jnp.max(x, axis=1, keepdims=True)
    col = lax.broadcasted_iota(jnp.int32, (BR, NCLS), 1)
    pred = jnp.min(jnp.where(x == m, col, NCLS), axis=1)
    s = jnp.sum(jnp.exp(x - m), axis=1)
    conf_ref[0, 0, :] = 1.0 / s
    acc_ref[0, 0, :] = (pred == lab_ref[0, 0, :]).astype(jnp.float32)


def _ece_kernel(conf_ref, acc_ref, pos_ref, ilo_ref, ece_ref):
    conf = conf_ref[...]
    acc = acc_ref[...]
    bits = lax.bitcast_convert_type(conf, jnp.int32)

    ilo = ilo_ref[...]
    ranks = jnp.concatenate([ilo, ilo + 1], axis=1)
    need = ranks + 1

    lo0 = jnp.zeros((1, N_RANKS), jnp.int32)
    hi0 = jnp.full((1, N_RANKS), ONE_BITS, jnp.int32)

    def body(_, carry):
        lo, hi = carry
        mid = (lo + hi) // 2
        cnts = []
        for j in range(N_RANKS):
            mj = mid[0, j]
            cnts.append(jnp.sum((bits <= mj).astype(jnp.int32)))
        cnt = jnp.stack(cnts).reshape(1, N_RANKS)
        ge = cnt >= need
        return jnp.where(ge, lo, mid + 1), jnp.where(ge, mid, hi)

    lo, hi = lax.fori_loop(0, 30, body, (lo0, hi0))
    qvals = lax.bitcast_convert_type(hi, jnp.float32)
    s_lo = qvals[:, :16]
    s_hi = qvals[:, 16:]

    pos = pos_ref[...]
    delta = pos - ilo.astype(jnp.float32)
    bvals = s_lo + delta * (s_hi - s_lo)
    bvals = jnp.where(pos > float(NPT - 1), s_hi, bvals)

    ece = jnp.float32(0.0)
    for b in range(N_BINS):
        lo_b = bvals[0, b]
        up_b = bvals[0, b + 1]
        in_bin = (conf > lo_b) & (conf <= up_b)
        cnt = jnp.sum(in_bin.astype(jnp.float32))
        sa = jnp.sum(jnp.where(in_bin, acc, 0.0))
        sc = jnp.sum(jnp.where(in_bin, conf, 0.0))
        prop = cnt / float(NPT)
        safe = jnp.maximum(cnt, 1.0)
        term = jnp.abs(sc / safe - sa / safe) * prop
        ece = ece + jnp.where(prop > 0, term, 0.0)
    ece_ref[...] = jnp.reshape(ece, (1, 1))


def kernel(logits, labels):
    labels3 = labels.reshape(NBLK, 1, BR)
    conf3, acc3 = pl.pallas_call(
        _rowstats_kernel,
        grid=(NBLK,),
        in_specs=[
            pl.BlockSpec((BR, NCLS), lambda i: (i, 0)),
            pl.BlockSpec((1, 1, BR), lambda i: (i, 0, 0)),
        ],
        out_specs=[
            pl.BlockSpec((1, 1, BR), lambda i: (i, 0, 0)),
            pl.BlockSpec((1, 1, BR), lambda i: (i, 0, 0)),
        ],
        out_shape=[
            jax.ShapeDtypeStruct((NBLK, 1, BR), jnp.float32),
            jax.ShapeDtypeStruct((NBLK, 1, BR), jnp.float32),
        ],
    )(logits, labels3)

    conf = conf3.reshape(NBLK, BR)
    acc = acc3.reshape(NBLK, BR)

    pos = jnp.linspace(0.0, float(NPT), N_BINS + 1).reshape(1, N_BINS + 1)
    ilo = jnp.clip(jnp.floor(pos).astype(jnp.int32), 0, NPT - 2)

    ece = pl.pallas_call(
        _ece_kernel,
        out_shape=jax.ShapeDtypeStruct((1, 1), jnp.float32),
    )(conf, acc, pos, ilo)
    return ece.reshape(1)


# speedup vs baseline: 1.2117x; 1.2117x over previous
import jax
import jax.numpy as jnp
from jax import lax
from jax.experimental import pallas as pl
from jax.experimental.pallas import tpu as pltpu

N_BINS = 15
NPT = 100000
NCLS = 1000
BR = 1000
NBLK = NPT // BR
NQ = N_BINS + 1
LOG2E = 1.4426950408889634
LO_BITS = 0x3A800000
ONE_BITS = 0x3F800000
N_ITERS = 27
CH = 98
PADN = CH * 1024 - NPT


def _rowstats_kernel(x_ref, lab_ref, conf_ref, acc_ref):
    x = x_ref[...]
    m = jnp.max(x, axis=1, keepdims=True)
    col = lax.broadcasted_iota(jnp.int32, (1, NCLS), 1)
    at_lab = jnp.where(col == lab_ref[0, 0, :][:, None], x, -jnp.inf)
    v_at_label = jnp.max(at_lab, axis=1)
    s = jnp.sum(jnp.exp2(x * LOG2E - m * LOG2E), axis=1)
    conf_ref[0, 0, :] = 1.0 / s
    acc_ref[0, 0, :] = (v_at_label == m[:, 0]).astype(jnp.float32)


def _ece_kernel(conf_ref, acc_ref, pos_ref, ilo_ref, ece_ref):
    ilo = ilo_ref[...]

    def load_bits(c):
        return lax.bitcast_convert_type(conf_ref[c], jnp.int32)

    def search16(needs):
        los0 = tuple(jnp.int32(LO_BITS) for _ in range(NQ))
        his0 = tuple(jnp.int32(ONE_BITS) for _ in range(NQ))

        def body(_, carry):
            los, his = carry
            mids = tuple((l + h) // 2 for l, h in zip(los, his))

            def cbody(c, accs):
                d = load_bits(c)
                return tuple(a + (d <= mm).astype(jnp.int32)
                             for a, mm in zip(accs, mids))

            z = tuple(jnp.zeros((8, 128), jnp.int32) for _ in range(NQ))
            accs = lax.fori_loop(0, CH, cbody, z)
            cnts = [jnp.sum(a) for a in accs]
            ge = [c >= n for c, n in zip(cnts, needs)]
            nlo = tuple(jnp.where(g, l, m + 1)
                        for g, l, m in zip(ge, los, mids))
            nhi = tuple(jnp.where(g, m, h)
                        for g, m, h in zip(ge, mids, his))
            return nlo, nhi

        _, his = lax.fori_loop(0, N_ITERS, body, (los0, his0))
        return his

    needA = [ilo[0, r] + 1 for r in range(NQ)]
    needB = [ilo[0, r] + 2 for r in range(NQ)]
    bitsA = search16(needA)
    bitsB = search16(needB)

    qA = jnp.stack(bitsA).reshape(1, NQ)
    qB = jnp.stack(bitsB).reshape(1, NQ)
    s_lo = lax.bitcast_convert_type(qA, jnp.float32)
    s_hi = lax.bitcast_convert_type(qB, jnp.float32)

    pos = pos_ref[...]
    delta = pos - ilo.astype(jnp.float32)
    bvals = s_lo + delta * (s_hi - s_lo)
    bvals = jnp.where(pos > float(NPT - 1), s_hi, bvals)
    bv = [bvals[0, j] for j in range(NQ)]

    def cum_counts():
        def cbody(c, accs):
            d = conf_ref[c]
            return tuple(a + (d <= b).astype(jnp.int32)
                         for a, b in zip(accs, bv))
        z = tuple(jnp.zeros((8, 128), jnp.int32) for _ in range(NQ))
        return lax.fori_loop(0, CH, cbody, z)

    def cum_masked(src_ref):
        def cbody(c, accs):
            d = conf_ref[c]
            v = src_ref[c]
            return tuple(a + jnp.where(d <= b, v, 0.0)
                         for a, b in zip(accs, bv))
        z = tuple(jnp.zeros((8, 128), jnp.float32) for _ in range(NQ))
        return lax.fori_loop(0, CH, cbody, z)

    ccnt = [jnp.sum(a).astype(jnp.float32) for a in cum_counts()]
    cacc = [jnp.sum(a) for a in cum_masked(acc_ref)]
    cconf = [jnp.sum(a) for a in cum_masked(conf_ref)]

    ece = jnp.float32(0.0)
    for b in range(N_BINS):
        cnt = ccnt[b + 1] - ccnt[b]
        sa = cacc[b + 1] - cacc[b]
        sc = cconf[b + 1] - cconf[b]
        prop = cnt / float(NPT)
        safe = jnp.maximum(cnt, 1.0)
        term = jnp.abs(sc / safe - sa / safe) * prop
        ece = ece + jnp.where(prop > 0, term, 0.0)
    ece_ref[...] = jnp.reshape(ece, (1, 1))


def kernel(logits, labels):
    labels3 = labels.reshape(NBLK, 1, BR)
    conf3, acc3 = pl.pallas_call(
        _rowstats_kernel,
        grid=(NBLK,),
        in_specs=[
            pl.BlockSpec((BR, NCLS), lambda i: (i, 0)),
            pl.BlockSpec((1, 1, BR), lambda i: (i, 0, 0)),
        ],
        out_specs=[
            pl.BlockSpec((1, 1, BR), lambda i: (i, 0, 0)),
            pl.BlockSpec((1, 1, BR), lambda i: (i, 0, 0)),
        ],
        out_shape=[
            jax.ShapeDtypeStruct((NBLK, 1, BR), jnp.float32),
            jax.ShapeDtypeStruct((NBLK, 1, BR), jnp.float32),
        ],
    )(logits, labels3)

    confp = jnp.concatenate(
        [conf3.reshape(-1), jnp.full((PADN,), 2.0, jnp.float32)]
    ).reshape(CH, 8, 128)
    accp = jnp.concatenate(
        [acc3.reshape(-1), jnp.zeros((PADN,), jnp.float32)]
    ).reshape(CH, 8, 128)

    pos = jnp.linspace(0.0, float(NPT), N_BINS + 1).reshape(1, N_BINS + 1)
    ilo = jnp.clip(jnp.floor(pos).astype(jnp.int32), 0, NPT - 2)

    ece = pl.pallas_call(
        _ece_kernel,
        out_shape=jax.ShapeDtypeStruct((1, 1), jnp.float32),
    )(confp, accp, pos, ilo)
    return ece.reshape(1)
